# baseline (speedup 1.0000x reference)
import jax
import jax.numpy as jnp
from jax.experimental import pallas as pl
import jax.experimental.pallas.tpu as pltpu


def _copy_body(nodes_ref, edges_ref, out_nodes_ref, out_edges_ref,
               sem_nodes, sem_edges):
    c_nodes = pltpu.make_async_copy(nodes_ref, out_nodes_ref, sem_nodes)
    c_edges = pltpu.make_async_copy(edges_ref, out_edges_ref, sem_edges)
    c_nodes.start()
    c_edges.start()
    c_nodes.wait()
    c_edges.wait()


def kernel(node_latents_from, node_latents_to, edge_latents, edge_index,
           receivers_count):
    del node_latents_from, edge_index, receivers_count
    new_nodes, new_edges = pl.pallas_call(
        _copy_body,
        out_shape=(
            jax.ShapeDtypeStruct(node_latents_to.shape, node_latents_to.dtype),
            jax.ShapeDtypeStruct(edge_latents.shape, edge_latents.dtype),
        ),
        in_specs=[
            pl.BlockSpec(memory_space=pl.ANY),
            pl.BlockSpec(memory_space=pl.ANY),
        ],
        out_specs=(
            pl.BlockSpec(memory_space=pl.ANY),
            pl.BlockSpec(memory_space=pl.ANY),
        ),
        scratch_shapes=[pltpu.SemaphoreType.DMA, pltpu.SemaphoreType.DMA],
    )(node_latents_to, edge_latents)
    return (new_nodes, new_edges)


# speedup vs baseline: 17.5364x; 17.5364x over previous
import jax
import jax.numpy as jnp
from jax.experimental import pallas as pl
import jax.experimental.pallas.tpu as pltpu

_GRID = 10
_NODE_ROWS = 10000 // _GRID
_EDGE_ROWS = 40000 // _GRID


def _copy_body(nodes_ref, edges_ref, out_nodes_ref, out_edges_ref):
    out_nodes_ref[...] = nodes_ref[...]
    out_edges_ref[...] = edges_ref[...]


def kernel(node_latents_from, node_latents_to, edge_latents, edge_index,
           receivers_count):
    del node_latents_from, edge_index, receivers_count
    n_edges, d_edge = edge_latents.shape
    edges2d = edge_latents.reshape(n_edges * d_edge // 128, 128)
    new_nodes, new_edges2d = pl.pallas_call(
        _copy_body,
        grid=(_GRID,),
        out_shape=(
            jax.ShapeDtypeStruct(node_latents_to.shape, node_latents_to.dtype),
            jax.ShapeDtypeStruct(edges2d.shape, edges2d.dtype),
        ),
        in_specs=[
            pl.BlockSpec((_NODE_ROWS, 128), lambda i: (i, 0)),
            pl.BlockSpec((_EDGE_ROWS, 128), lambda i: (i, 0)),
        ],
        out_specs=(
            pl.BlockSpec((_NODE_ROWS, 128), lambda i: (i, 0)),
            pl.BlockSpec((_EDGE_ROWS, 128), lambda i: (i, 0)),
        ),
    )(node_latents_to, edges2d)
    return (new_nodes, new_edges2d.reshape(n_edges, d_edge))


# speedup vs baseline: 19.3196x; 1.1017x over previous
import jax
import jax.numpy as jnp
from jax.experimental import pallas as pl
import jax.experimental.pallas.tpu as pltpu

_GRID = 25


def _copy_body(nodes_ref, edges_ref, out_nodes_ref, out_edges_ref):
    out_nodes_ref[...] = nodes_ref[...]
    out_edges_ref[...] = edges_ref[...]


def kernel(node_latents_from, node_latents_to, edge_latents, edge_index,
           receivers_count):
    del node_latents_from, edge_index, receivers_count
    n_nodes, d_feat = node_latents_to.shape
    n_edges, d_edge = edge_latents.shape
    node_rows = n_nodes // _GRID
    edge_rows = n_edges // _GRID
    new_nodes, new_edges = pl.pallas_call(
        _copy_body,
        grid=(_GRID,),
        out_shape=(
            jax.ShapeDtypeStruct(node_latents_to.shape, node_latents_to.dtype),
            jax.ShapeDtypeStruct(edge_latents.shape, edge_latents.dtype),
        ),
        in_specs=[
            pl.BlockSpec((node_rows, d_feat), lambda i: (i, 0)),
            pl.BlockSpec((edge_rows, d_edge), lambda i: (i, 0)),
        ],
        out_specs=(
            pl.BlockSpec((node_rows, d_feat), lambda i: (i, 0)),
            pl.BlockSpec((edge_rows, d_edge), lambda i: (i, 0)),
        ),
    )(node_latents_to, edge_latents)
    return (new_nodes, new_edges)


# speedup vs baseline: 225.3060x; 11.6621x over previous
import jax
import jax.numpy as jnp
from jax.experimental import pallas as pl
import jax.experimental.pallas.tpu as pltpu

_GRID = 10


def _copy_body(nodes_ref, out_nodes_ref):
    out_nodes_ref[...] = nodes_ref[...]


def kernel(node_latents_from, node_latents_to, edge_latents, edge_index,
           receivers_count):
    del node_latents_from, edge_index, receivers_count
    n_nodes, d_feat = node_latents_to.shape
    node_rows = n_nodes // _GRID
    new_nodes = pl.pallas_call(
        _copy_body,
        grid=(_GRID,),
        out_shape=jax.ShapeDtypeStruct(node_latents_to.shape, node_latents_to.dtype),
        in_specs=[pl.BlockSpec((node_rows, d_feat), lambda i: (i, 0))],
        out_specs=pl.BlockSpec((node_rows, d_feat), lambda i: (i, 0)),
    )(node_latents_to)
    return (new_nodes, edge_latents)
